# baseline (speedup 1.0000x reference)
import functools

import jax
import jax.numpy as jnp
from jax import lax
from jax.experimental import pallas as pl
from jax.experimental.pallas import tpu as pltpu
from jax.experimental.pallas import tpu_sc as plsc

N = 10000
E = 320000
D = 128
G = 128
NC = 2
NS = 16
NGRP = 2560
GP_TEC = NGRP // (NC * NS)
EPAD = NGRP * G
RPS = N // NS


def _zero_rows(rows_v, n_rows, width):
    z16 = jnp.zeros((16,), jnp.float32)

    def body(i, _):
        for j in range(width // 16):
            rows_v[i, pl.ds(j * 16, 16)] = z16
        return 0

    lax.fori_loop(0, n_rows, body, 0)


def _zero_acc_slice(rows_v, acc, base):
    for k in range(4):
        pltpu.sync_copy(rows_v, acc.at[pl.ds(base + k * G, G)])
    pltpu.sync_copy(rows_v.at[pl.ds(0, RPS - 4 * G)],
                    acc.at[pl.ds(base + 4 * G, RPS - 4 * G)])


def _spmm_body(pk_hbm, x_hbm, out_hbm,
               ib0, ib1, sr0, sr1, rw0, rw1, acc,
               si0, si1, sg0, sg1, sh0, sh1, ss0, ss1):
    c = lax.axis_index("c")
    s = lax.axis_index("s")
    ib = (ib0, ib1)
    sr = (sr0, sr1)
    rw = (rw0, rw1)
    si = (si0, si1)
    sg = (sg0, sg1)
    sg2 = (sh0, sh1)
    ss = (ss0, ss1)

    _zero_rows(rw0, G, D)
    _zero_acc_slice(rw0, acc, s * RPS)
    plsc.subcore_barrier()

    g0 = (c * NS + s) * GP_TEC

    def issue_idx(g, p):
        grp = jnp.minimum(g0 + g, NGRP - 1)
        pltpu.async_copy(pk_hbm.at[grp], ib[p], si[p])

    def issue_gather(g, p):
        del g
        idx = ib[p].at[0]
        pltpu.async_copy(x_hbm.at[idx.at[pl.ds(0, G // 2)]],
                         rw[p].at[pl.ds(0, G // 2)], sg[p])
        pltpu.async_copy(x_hbm.at[idx.at[pl.ds(G // 2, G // 2)]],
                         rw[p].at[pl.ds(G // 2, G // 2)], sg2[p])

    def compute(g, p):
        del g
        ibp = ib[p]
        rwp = rw[p]
        iota16 = lax.iota(jnp.int32, 16)
        for ch in range(G // 16):
            off = ch * 16
            sr[p][pl.ds(off, 16)] = ibp[1, pl.ds(off, 16)]

        @plsc.parallel_loop(0, G, 1, unroll=8)
        def _row(r):
            base = (r >> 4) << 4
            v16 = plsc.bitcast(ibp[2, pl.ds(base, 16)], jnp.float32)
            rl = r & 15
            splat = jnp.broadcast_to(
                jnp.sum(jnp.where(iota16 == rl, v16, 0.0)), (16,))
            for j in range(D // 16):
                rwp[r, pl.ds(j * 16, 16)] = \
                    rwp[r, pl.ds(j * 16, 16)] * splat

    def issue_scatter(g, p):
        del g
        pltpu.async_copy(rw[p], acc.at[sr[p]], ss[p], add=True)

    def wait_idx(p):
        pltpu.make_async_copy(pk_hbm.at[0], ib[p], si[p]).wait()

    def wait_gather(p):
        idx = ib[p].at[0]
        pltpu.make_async_copy(x_hbm.at[idx.at[pl.ds(0, G // 2)]],
                              rw[p].at[pl.ds(0, G // 2)], sg[p]).wait()
        pltpu.make_async_copy(x_hbm.at[idx.at[pl.ds(G // 2, G // 2)]],
                              rw[p].at[pl.ds(G // 2, G // 2)], sg2[p]).wait()

    def wait_scatter(p):
        pltpu.make_async_copy(rw[p], acc.at[sr[p]], ss[p]).wait()

    issue_idx(0, 0)
    issue_idx(1, 1)

    def step(g, p, first):
        wait_gather(p)
        if not first:
            wait_scatter(1 - p)
        wait_idx(1 - p)
        issue_gather(g + 1, 1 - p)
        compute(g, p)
        issue_scatter(g, p)
        issue_idx(g + 2, p)

    wait_idx(0)
    issue_gather(0, 0)
    step(0, 0, True)

    def pair_body(k, _):
        g = 2 * k + 1
        step(g, 1, False)
        step(g + 1, 0, False)
        return 0

    lax.fori_loop(0, (GP_TEC - 2) // 2, pair_body, 0)

    gl = GP_TEC - 1
    wait_gather(1)
    wait_scatter(0)
    wait_idx(0)
    compute(gl, 1)
    issue_scatter(gl, 1)
    wait_scatter(1)

    plsc.subcore_barrier()
    base = s * RPS
    pltpu.sync_copy(acc.at[pl.ds(base, RPS)], out_hbm.at[c, pl.ds(base, RPS)])


def _make_spmm():
    mesh = plsc.VectorSubcoreMesh(core_axis_name="c", subcore_axis_name="s",
                                  num_cores=NC, num_subcores=NS)
    scratch = [
        pltpu.VMEM((3, G), jnp.int32),
        pltpu.VMEM((3, G), jnp.int32),
        pltpu.VMEM((G,), jnp.int32),
        pltpu.VMEM((G,), jnp.int32),
        pltpu.VMEM((G, D), jnp.float32),
        pltpu.VMEM((G, D), jnp.float32),
        pltpu.VMEM_SHARED((N, D), jnp.float32),
        pltpu.SemaphoreType.DMA,
        pltpu.SemaphoreType.DMA,
        pltpu.SemaphoreType.DMA,
        pltpu.SemaphoreType.DMA,
        pltpu.SemaphoreType.DMA,
        pltpu.SemaphoreType.DMA,
        pltpu.SemaphoreType.DMA,
        pltpu.SemaphoreType.DMA,
    ]
    return pl.kernel(
        _spmm_body,
        out_type=jax.ShapeDtypeStruct((NC, N, D), jnp.float32),
        mesh=mesh,
        scratch_types=scratch,
        compiler_params=pltpu.CompilerParams(use_tc_tiling_on_sc=False,
                                             needs_layout_passes=False),
        name="spmm_edges",
    )


def _deg_body(pk_hbm, out_hbm, ib_v, buf_v, acc, sem):
    c = lax.axis_index("c")
    s = lax.axis_index("s")
    _zero_rows(buf_v, G, 16)
    _zero_acc_slice(buf_v, acc, s * RPS)
    plsc.subcore_barrier()

    g0 = (c * NS + s) * GP_TEC
    iota = lax.iota(jnp.int32, 16)
    zcol = jnp.zeros((16,), jnp.int32)

    def grp_body(g, _):
        pltpu.sync_copy(pk_hbm.at[g0 + g], ib_v)
        for ch in range(G // 16):
            off = ch * 16
            v16 = plsc.bitcast(ib_v[2, pl.ds(off, 16)], jnp.float32)
            plsc.store_scatter(buf_v, [iota + off, zcol], v16)
        pltpu.sync_copy(buf_v, acc.at[ib_v.at[1]], add=True)
        return 0

    lax.fori_loop(0, GP_TEC, grp_body, 0)
    plsc.subcore_barrier()
    base = s * RPS
    pltpu.sync_copy(acc.at[pl.ds(base, RPS)], out_hbm.at[c, pl.ds(base, RPS)])


def _make_deg():
    mesh = plsc.VectorSubcoreMesh(core_axis_name="c", subcore_axis_name="s",
                                  num_cores=NC, num_subcores=NS)
    scratch = [
        pltpu.VMEM((3, G), jnp.int32),
        pltpu.VMEM((G, 16), jnp.float32),
        pltpu.VMEM_SHARED((N, 16), jnp.float32),
        pltpu.SemaphoreType.DMA,
    ]
    return pl.kernel(
        _deg_body,
        out_type=jax.ShapeDtypeStruct((NC, N, 16), jnp.float32),
        mesh=mesh,
        scratch_types=scratch,
        compiler_params=pltpu.CompilerParams(use_tc_tiling_on_sc=False,
                                             needs_layout_passes=False),
        name="deg_seg_sum",
    )


@functools.lru_cache(maxsize=None)
def _get_kernels():
    return _make_spmm(), _make_deg()


def _pack_edges(indices, values):
    pad = EPAD - E
    col = jnp.concatenate([indices[1].astype(jnp.int32),
                           jnp.zeros((pad,), jnp.int32)]).reshape(NGRP, 1, G)
    row = jnp.concatenate([indices[0].astype(jnp.int32),
                           jnp.zeros((pad,), jnp.int32)]).reshape(NGRP, 1, G)
    vbits = lax.bitcast_convert_type(
        jnp.concatenate([values, jnp.zeros((pad,), jnp.float32)]),
        jnp.int32).reshape(NGRP, 1, G)
    return jnp.concatenate([col, row, vbits], axis=1)


def kernel(x, shuf, adj_indices, adj_values, neighbor_indices,
           neighbor_values, temp):
    _spmm, _deg_kernel = _get_kernels()
    a_pk = _pack_edges(adj_indices, adj_values)
    n_pk = _pack_edges(neighbor_indices, neighbor_values)

    degp = _deg_kernel(a_pk)
    deg = degp[0, :, 0] + degp[1, :, 0]
    dis = jnp.where(deg > 0, lax.rsqrt(jnp.where(deg > 0, deg, 1.0)),
                    0.0)[:, None]

    u1p = _spmm(a_pk, dis * x)
    u1 = dis * (u1p[0] + u1p[1])
    u2p = _spmm(a_pk, dis * u1)
    u2 = dis * (u2p[0] + u2p[1])

    T = jax.nn.relu(temp)
    a = T[0]
    b = T[1] - T[0]
    c4 = (T[0] + T[2] - 2.0 * T[1]) / 4.0
    out = (a + b + c4) * x - (b + 2.0 * c4) * u1 + c4 * u2

    tp = _spmm(n_pk, out)
    zp = _spmm(n_pk, tp[0] + tp[1])
    z_pos = zp[0] + zp[1]

    tn = _spmm(n_pk, out[shuf, :])
    zn = _spmm(n_pk, tn[0] + tn[1])
    z_neg = zn[0] + zn[1]

    return out, z_pos, z_neg


# speedup vs baseline: 2.3430x; 2.3430x over previous
import functools

import jax
import jax.numpy as jnp
from jax import lax
from jax.experimental import pallas as pl
from jax.experimental.pallas import tpu as pltpu
from jax.experimental.pallas import tpu_sc as plsc

N = 10000
E = 320000
D = 128
DH = D // 2
G = 128
NC = 2
NS = 16
NGRP = 2560
GP_TEC = NGRP // NS
EPAD = NGRP * G
RPS = N // NS


def _zero_rows(rows_v, n_rows, width):
    z16 = jnp.zeros((16,), jnp.float32)

    def body(i, _):
        for j in range(width // 16):
            rows_v[i, pl.ds(j * 16, 16)] = z16
        return 0

    lax.fori_loop(0, n_rows, body, 0)


def _zero_acc_slice(rows_v, acc, base):
    for k in range(4):
        pltpu.sync_copy(rows_v, acc.at[pl.ds(base + k * G, G)])
    pltpu.sync_copy(rows_v.at[pl.ds(0, RPS - 4 * G)],
                    acc.at[pl.ds(base + 4 * G, RPS - 4 * G)])


def _spmm_body(pk_hbm, xs_hbm, out_hbm,
               ib0, ib1, sr0, sr1, rw0, rw1, xtab, acc,
               si0, si1, sg0, sg1, ss0, ss1):
    c = lax.axis_index("c")
    s = lax.axis_index("s")
    ib = (ib0, ib1)
    sr = (sr0, sr1)
    rw = (rw0, rw1)
    si = (si0, si1)
    sg = (sg0, sg1)
    ss = (ss0, ss1)

    base = s * RPS
    _zero_rows(rw0, G, DH)
    _zero_acc_slice(rw0, acc, base)
    pltpu.sync_copy(xs_hbm.at[c, pl.ds(base, RPS)], xtab.at[pl.ds(base, RPS)])
    plsc.subcore_barrier()

    g0 = s * GP_TEC

    def issue_idx(g, p):
        grp = jnp.minimum(g0 + g, NGRP - 1)
        pltpu.async_copy(pk_hbm.at[grp], ib[p], si[p])

    def issue_gather(g, p):
        del g
        pltpu.async_copy(xtab.at[ib[p].at[0]], rw[p], sg[p])

    def compute(g, p):
        del g
        ibp = ib[p]
        rwp = rw[p]
        iota16 = lax.iota(jnp.int32, 16)
        for ch in range(G // 16):
            off = ch * 16
            sr[p][pl.ds(off, 16)] = ibp[1, pl.ds(off, 16)]

        @plsc.parallel_loop(0, G, 1, unroll=8)
        def _row(r):
            bs = (r >> 4) << 4
            v16 = plsc.bitcast(ibp[2, pl.ds(bs, 16)], jnp.float32)
            rl = r & 15
            splat = jnp.broadcast_to(
                jnp.sum(jnp.where(iota16 == rl, v16, 0.0)), (16,))
            for j in range(DH // 16):
                rwp[r, pl.ds(j * 16, 16)] = \
                    rwp[r, pl.ds(j * 16, 16)] * splat

    def issue_scatter(g, p):
        del g
        pltpu.async_copy(rw[p], acc.at[sr[p]], ss[p], add=True)

    def wait_idx(p):
        pltpu.make_async_copy(pk_hbm.at[0], ib[p], si[p]).wait()

    def wait_gather(p):
        pltpu.make_async_copy(xtab.at[ib[p].at[0]], rw[p], sg[p]).wait()

    def wait_scatter(p):
        pltpu.make_async_copy(rw[p], acc.at[sr[p]], ss[p]).wait()

    issue_idx(0, 0)
    issue_idx(1, 1)

    def step(g, p, first):
        wait_gather(p)
        if not first:
            wait_scatter(1 - p)
        wait_idx(1 - p)
        issue_gather(g + 1, 1 - p)
        compute(g, p)
        issue_scatter(g, p)
        issue_idx(g + 2, p)

    wait_idx(0)
    issue_gather(0, 0)
    step(0, 0, True)

    def pair_body(k, _):
        g = 2 * k + 1
        step(g, 1, False)
        step(g + 1, 0, False)
        return 0

    lax.fori_loop(0, (GP_TEC - 2) // 2, pair_body, 0)

    gl = GP_TEC - 1
    wait_gather(1)
    wait_scatter(0)
    wait_idx(0)
    compute(gl, 1)
    issue_scatter(gl, 1)
    wait_scatter(1)

    plsc.subcore_barrier()
    pltpu.sync_copy(acc.at[pl.ds(base, RPS)], out_hbm.at[c, pl.ds(base, RPS)])


def _make_spmm():
    mesh = plsc.VectorSubcoreMesh(core_axis_name="c", subcore_axis_name="s",
                                  num_cores=NC, num_subcores=NS)
    scratch = [
        pltpu.VMEM((3, G), jnp.int32),
        pltpu.VMEM((3, G), jnp.int32),
        pltpu.VMEM((G,), jnp.int32),
        pltpu.VMEM((G,), jnp.int32),
        pltpu.VMEM((G, DH), jnp.float32),
        pltpu.VMEM((G, DH), jnp.float32),
        pltpu.VMEM_SHARED((N, DH), jnp.float32),
        pltpu.VMEM_SHARED((N, DH), jnp.float32),
        pltpu.SemaphoreType.DMA,
        pltpu.SemaphoreType.DMA,
        pltpu.SemaphoreType.DMA,
        pltpu.SemaphoreType.DMA,
        pltpu.SemaphoreType.DMA,
        pltpu.SemaphoreType.DMA,
    ]
    return pl.kernel(
        _spmm_body,
        out_type=jax.ShapeDtypeStruct((NC, N, DH), jnp.float32),
        mesh=mesh,
        scratch_types=scratch,
        compiler_params=pltpu.CompilerParams(use_tc_tiling_on_sc=False,
                                             needs_layout_passes=False),
        name="spmm_edges",
    )


def _deg_body(pk_hbm, out_hbm, ib_v, buf_v, acc, sem):
    c = lax.axis_index("c")
    s = lax.axis_index("s")
    _zero_rows(buf_v, G, 16)
    _zero_acc_slice(buf_v, acc, s * RPS)
    plsc.subcore_barrier()

    g0 = (c * NS + s) * (NGRP // (NC * NS))
    iota = lax.iota(jnp.int32, 16)
    zcol = jnp.zeros((16,), jnp.int32)

    def grp_body(g, _):
        pltpu.sync_copy(pk_hbm.at[g0 + g], ib_v)
        for ch in range(G // 16):
            off = ch * 16
            v16 = plsc.bitcast(ib_v[2, pl.ds(off, 16)], jnp.float32)
            plsc.store_scatter(buf_v, [iota + off, zcol], v16)
        pltpu.sync_copy(buf_v, acc.at[ib_v.at[1]], add=True)
        return 0

    lax.fori_loop(0, NGRP // (NC * NS), grp_body, 0)
    plsc.subcore_barrier()
    base = s * RPS
    pltpu.sync_copy(acc.at[pl.ds(base, RPS)], out_hbm.at[c, pl.ds(base, RPS)])


def _make_deg():
    mesh = plsc.VectorSubcoreMesh(core_axis_name="c", subcore_axis_name="s",
                                  num_cores=NC, num_subcores=NS)
    scratch = [
        pltpu.VMEM((3, G), jnp.int32),
        pltpu.VMEM((G, 16), jnp.float32),
        pltpu.VMEM_SHARED((N, 16), jnp.float32),
        pltpu.SemaphoreType.DMA,
    ]
    return pl.kernel(
        _deg_body,
        out_type=jax.ShapeDtypeStruct((NC, N, 16), jnp.float32),
        mesh=mesh,
        scratch_types=scratch,
        compiler_params=pltpu.CompilerParams(use_tc_tiling_on_sc=False,
                                             needs_layout_passes=False),
        name="deg_seg_sum",
    )


@functools.lru_cache(maxsize=None)
def _get_kernels():
    return _make_spmm(), _make_deg()


def _pack_edges(indices, values):
    pad = EPAD - E
    col = jnp.concatenate([indices[1].astype(jnp.int32),
                           jnp.zeros((pad,), jnp.int32)]).reshape(NGRP, 1, G)
    row = jnp.concatenate([indices[0].astype(jnp.int32),
                           jnp.zeros((pad,), jnp.int32)]).reshape(NGRP, 1, G)
    vbits = lax.bitcast_convert_type(
        jnp.concatenate([values, jnp.zeros((pad,), jnp.float32)]),
        jnp.int32).reshape(NGRP, 1, G)
    return jnp.concatenate([col, row, vbits], axis=1)


def _split(y):
    return y.reshape(N, NC, DH).swapaxes(0, 1)


def _unsplit(ys):
    return ys.swapaxes(0, 1).reshape(N, D)


def kernel(x, shuf, adj_indices, adj_values, neighbor_indices,
           neighbor_values, temp):
    _spmm, _deg_kernel = _get_kernels()
    a_pk = _pack_edges(adj_indices, adj_values)
    n_pk = _pack_edges(neighbor_indices, neighbor_values)

    degp = _deg_kernel(a_pk)
    deg = degp[0, :, 0] + degp[1, :, 0]
    dis = jnp.where(deg > 0, lax.rsqrt(jnp.where(deg > 0, deg, 1.0)),
                    0.0)[:, None]

    xs = _split(x)
    u1 = dis * _spmm(a_pk, _split(dis * x))
    u2 = dis * _spmm(a_pk, dis * u1)

    T = jax.nn.relu(temp)
    a = T[0]
    b = T[1] - T[0]
    c4 = (T[0] + T[2] - 2.0 * T[1]) / 4.0
    outs = (a + b + c4) * xs - (b + 2.0 * c4) * u1 + c4 * u2

    zp = _spmm(n_pk, _spmm(n_pk, outs))
    zn = _spmm(n_pk, _spmm(n_pk, outs[:, shuf, :]))

    return _unsplit(outs), _unsplit(zp), _unsplit(zn)
